# baseline (speedup 1.0000x reference)
import functools

import jax
import jax.numpy as jnp
from jax import lax
from jax.experimental import pallas as pl
from jax.experimental.pallas import tpu as pltpu
from jax.experimental.pallas import tpu_sc as plsc

NC = 2
NS = 16
NW = NC * NS
LANES = 16
NBUF = 3
TC_BLK = 1024


def _sc_add(x, pe):
    seq, d = x.shape
    rows_per_w = seq // NW
    chunk = min(rows_per_w, max(1, 8192 // d))
    n_chunks = rows_per_w // chunk
    vecs_per_row = d // LANES

    mesh = plsc.VectorSubcoreMesh(core_axis_name="c", subcore_axis_name="s")

    @functools.partial(
        pl.kernel,
        out_type=jax.ShapeDtypeStruct((seq, d), jnp.float32),
        mesh=mesh,
        scratch_types=[
            pltpu.VMEM((NBUF, chunk, d), jnp.float32),
            pltpu.VMEM((NBUF, chunk, d), jnp.float32),
            pltpu.SemaphoreType.DMA((NBUF,)),
            pltpu.SemaphoreType.DMA((NBUF,)),
        ],
    )
    def k(x_hbm, p_hbm, o_hbm, x_v, p_v, ld_sem, st_sem):
        wid = lax.axis_index("s") * NC + lax.axis_index("c")
        base = wid * rows_per_w

        loads = {}
        stores = {}

        def start_load(c):
            b = c % NBUF
            off = base + c * chunk
            loads[c] = (
                pltpu.make_async_copy(
                    x_hbm.at[pl.ds(off, chunk), :], x_v.at[b], ld_sem.at[b]
                ),
                pltpu.make_async_copy(
                    p_hbm.at[pl.ds(off, chunk), :], p_v.at[b], ld_sem.at[b]
                ),
            )
            loads[c][0].start()
            loads[c][1].start()

        for c in range(min(NBUF, n_chunks)):
            start_load(c)

        for c in range(n_chunks):
            b = c % NBUF
            for dsc in loads.pop(c):
                dsc.wait()

            @plsc.parallel_loop(0, chunk, unroll=4)
            def _(r):
                for j in range(vecs_per_row):
                    s = pl.ds(j * LANES, LANES)
                    x_v[b, r, s] = x_v[b, r, s] + p_v[b, r, s]

            off = base + c * chunk
            stores[c] = pltpu.make_async_copy(
                x_v.at[b], o_hbm.at[pl.ds(off, chunk), :], st_sem.at[b]
            )
            stores[c].start()

            nxt = c + NBUF
            if nxt < n_chunks:
                stores.pop(nxt - NBUF).wait()
                start_load(nxt)

        for dsc in stores.values():
            dsc.wait()

    return k(x, pe)


def _tc_add_bottom(x, pe, h):
    seq, d = x.shape
    n_blk = (seq - h) // TC_BLK
    off_blk = h // TC_BLK

    def body(x_ref, pe_ref, o_ref):
        o_ref[...] = x_ref[...] + pe_ref[...]

    return pl.pallas_call(
        body,
        grid=(n_blk,),
        in_specs=[
            pl.BlockSpec((TC_BLK, d), lambda i: (i + off_blk, 0)),
            pl.BlockSpec((TC_BLK, d), lambda i: (i + off_blk, 0)),
        ],
        out_specs=pl.BlockSpec((TC_BLK, d), lambda i: (i, 0)),
        out_shape=jax.ShapeDtypeStruct((seq - h, d), x.dtype),
    )(x, pe)


@jax.jit
def _combined(x, pe):
    seq, d = x.shape
    pe = pe[:seq]
    h = seq // 2
    w = h * d // 128
    xs = x[:h].reshape(w, 128)
    ps = pe[:h].reshape(w, 128)
    top = _sc_add(xs, ps)
    bottom = _tc_add_bottom(x, pe, h)
    return jnp.concatenate([top.reshape(h, d), bottom], axis=0)


def kernel(x, pe):
    return _combined(x, pe)


# speedup vs baseline: 1.5753x; 1.5753x over previous
import functools

import jax
import jax.numpy as jnp
from jax import lax
from jax.experimental import pallas as pl
from jax.experimental.pallas import tpu as pltpu
from jax.experimental.pallas import tpu_sc as plsc

NC = 2
NS = 16
NW = NC * NS
LANES = 16
NBUF = 4


@jax.jit
def _sc_add(x, pe):
    seq, d = x.shape
    rows_per_w = seq // NW
    chunk = min(rows_per_w, max(1, 8192 // d))
    n_chunks = rows_per_w // chunk
    vecs_per_row = d // LANES

    mesh = plsc.VectorSubcoreMesh(core_axis_name="c", subcore_axis_name="s")

    @functools.partial(
        pl.kernel,
        out_type=jax.ShapeDtypeStruct((seq, d), jnp.float32),
        mesh=mesh,
        scratch_types=[
            pltpu.VMEM((NBUF, chunk, d), jnp.float32),
            pltpu.VMEM((NBUF, chunk, d), jnp.float32),
            pltpu.SemaphoreType.DMA((NBUF,)),
            pltpu.SemaphoreType.DMA((NBUF,)),
            pltpu.SemaphoreType.DMA((NBUF,)),
        ],
    )
    def k(x_hbm, p_hbm, o_hbm, x_v, p_v, ldx_sem, ldp_sem, st_sem):
        wid = lax.axis_index("s") * NC + lax.axis_index("c")
        base = wid * rows_per_w

        loads = {}
        stores = {}

        def start_load(c):
            b = c % NBUF
            off = base + c * chunk
            loads[c] = (
                pltpu.make_async_copy(
                    x_hbm.at[pl.ds(off, chunk), :], x_v.at[b], ldx_sem.at[b]
                ),
                pltpu.make_async_copy(
                    p_hbm.at[pl.ds(off, chunk), :], p_v.at[b], ldp_sem.at[b]
                ),
            )
            loads[c][0].start()
            loads[c][1].start()

        for c in range(min(NBUF, n_chunks)):
            start_load(c)

        for c in range(n_chunks):
            b = c % NBUF
            for dsc in loads.pop(c):
                dsc.wait()

            @plsc.parallel_loop(0, chunk, unroll=8)
            def _(r):
                for j in range(vecs_per_row):
                    s = pl.ds(j * LANES, LANES)
                    x_v[b, r, s] = x_v[b, r, s] + p_v[b, r, s]

            off = base + c * chunk
            stores[c] = pltpu.make_async_copy(
                x_v.at[b], o_hbm.at[pl.ds(off, chunk), :], st_sem.at[b]
            )
            stores[c].start()

            nxt = c + NBUF
            if nxt < n_chunks:
                stores.pop(nxt - NBUF).wait()
                start_load(nxt)

        for dsc in stores.values():
            dsc.wait()

    return k(x, pe)


def kernel(x, pe):
    return _sc_add(x, pe[: x.shape[0]])
